# baseline (speedup 1.0000x reference)
import functools

import jax
import jax.numpy as jnp
from jax import lax
from jax.experimental import pallas as pl
from jax.experimental.pallas import tpu as pltpu
from jax.experimental.pallas import tpu_sc as plsc

NC = 2
NS = 16
NW = NC * NS
B = 128
RING = 4
PF = 2

F32 = jnp.float32


def _mesh():
  return plsc.VectorSubcoreMesh(core_axis_name="c", subcore_axis_name="s")


def _fill(buf, nrows, ncols, value):
  v = jnp.full((16,), value, F32)
  per_row = ncols // 16

  def body(i, _):
    buf[i // per_row, pl.ds((i % per_row) * 16, 16)] = v
    return 0

  lax.fori_loop(0, nrows * per_row, body, 0)


def _deg_body(nb, rows_per_tile,
              src2, dst2, zeros_hbm, deg_s_out, deg_d_out,
              idx_s, idx_d, ones_v, acc, sem):
  c = lax.axis_index("c")
  s = lax.axis_index("s")
  w = c * NS + s

  _fill(ones_v, B, 16, 1.0)
  base = s * rows_per_tile

  pltpu.sync_copy(src2.at[pl.ds(w * nb, nb)], idx_s)
  pltpu.sync_copy(dst2.at[pl.ds(w * nb, nb)], idx_d)

  for idx, out in ((idx_s, deg_s_out), (idx_d, deg_d_out)):
    pltpu.sync_copy(zeros_hbm, acc.at[pl.ds(base, rows_per_tile)])
    plsc.subcore_barrier()

    def edge_body(j, _, idx=idx):
      pltpu.async_copy(ones_v, acc.at[idx.at[j]], sem, add=True)
      return 0

    lax.fori_loop(0, nb, edge_body, 0)

    def drain_body(j, _, idx=idx):
      pltpu.make_async_copy(ones_v, acc.at[idx.at[0]], sem).wait()
      return 0

    lax.fori_loop(0, nb, drain_body, 0)
    plsc.subcore_barrier()

    pltpu.sync_copy(acc.at[pl.ds(base, rows_per_tile)],
                    out.at[c, pl.ds(base, rows_per_tile)])
    plsc.subcore_barrier()


def _deg_call(np_pad, nb, src2, dst2, zeros_hbm):
  rows_per_tile = np_pad // NS
  out = jax.ShapeDtypeStruct((NC, np_pad, 16), F32)
  f = pl.kernel(
      functools.partial(_deg_body, nb, rows_per_tile),
      out_type=[out, out],
      mesh=_mesh(),
      scratch_types=[
          pltpu.VMEM((nb, B), jnp.int32),
          pltpu.VMEM((nb, B), jnp.int32),
          pltpu.VMEM((B, 16), F32),
          pltpu.VMEM_SHARED((np_pad, 16), F32),
          pltpu.SemaphoreType.DMA,
      ],
  )
  return f(src2, dst2, zeros_hbm)


def _prop_body(nbt, quarter, rows_per_tile,
               table, src2, dst2, zeros_hbm, part_out,
               idx_s, idx_d, rows, acc, gsem, ssem):
  c = lax.axis_index("c")
  s = lax.axis_index("s")
  base = s * rows_per_tile

  pltpu.sync_copy(src2.at[pl.ds(s * nbt, nbt)], idx_s)

  def gather(j, b):
    pltpu.async_copy(table.at[idx_s.at[j]], rows.at[b], gsem.at[b])

  def gather_wait(j, b):
    pltpu.make_async_copy(table.at[idx_s.at[j]], rows.at[b],
                          gsem.at[b]).wait()

  def scatter(j, b):
    pltpu.async_copy(rows.at[b], acc.at[idx_d.at[j]], ssem.at[b], add=True)

  def scatter_wait(b):
    pltpu.make_async_copy(rows.at[b], acc.at[idx_d.at[0]],
                          ssem.at[b]).wait()

  for qp in range(2):
    q = c * 2 + qp

    pltpu.sync_copy(zeros_hbm, acc.at[pl.ds(base, rows_per_tile)])

    pltpu.sync_copy(dst2.at[pl.ds(s * nbt, nbt)], idx_d)
    qbase = q * quarter

    dummy = quarter + lax.broadcasted_iota(jnp.int32, (16,), 0)

    def remap(i, _, qbase=qbase, dummy=dummy):
      r = i // 8
      k = (i % 8) * 16
      v = idx_d[r, pl.ds(k, 16)] - qbase
      ok = (v >= 0) & (v < quarter)
      idx_d[r, pl.ds(k, 16)] = jnp.where(ok, v, dummy)
      return 0

    lax.fori_loop(0, nbt * 8, remap, 0)
    plsc.subcore_barrier()

    for b in range(PF):
      gather(b, b)

    def group(g, _):
      for b in range(RING):
        j = g * RING + b
        jp = j + PF
        bp = (b + PF) % RING

        @pl.when(jp < nbt)
        def _():
          @pl.when(jp >= RING)
          def _():
            scatter_wait(bp)
          gather(jp, bp)

        gather_wait(j, b)
        scatter(j, b)
      return 0

    lax.fori_loop(0, nbt // RING, group, 0)
    for b in range(RING):
      scatter_wait(b)
    plsc.subcore_barrier()

    pltpu.sync_copy(acc.at[pl.ds(base, rows_per_tile)],
                    part_out.at[q, pl.ds(base, rows_per_tile)])
    plsc.subcore_barrier()


def _prop_call(np_pad, nbatches, table, src2, dst2, zeros_hbm):
  quarter = np_pad // (2 * NC)
  rows_per_tile = quarter // NS
  nbt = nbatches // NS
  f = pl.kernel(
      functools.partial(_prop_body, nbt, quarter, rows_per_tile),
      out_type=jax.ShapeDtypeStruct((2 * NC, quarter, 128), F32),
      mesh=_mesh(),
      scratch_types=[
          pltpu.VMEM((nbt, B), jnp.int32),
          pltpu.VMEM((nbt, B), jnp.int32),
          pltpu.VMEM((RING, B, 128), F32),
          pltpu.VMEM_SHARED((quarter + 16, 128), F32),
          pltpu.SemaphoreType.DMA((RING,)),
          pltpu.SemaphoreType.DMA((RING,)),
      ],
  )
  return f(table, src2, dst2, zeros_hbm)


def _tc1_body(xp_ref, ds_ref, dd_ref, w1_ref, hw1_ref, ns_ref, nd_ref):
  deg_s = ds_ref[0, :, 0:1] + ds_ref[1, :, 0:1] + 1.0
  deg_d = dd_ref[0, :, 0:1] + dd_ref[1, :, 0:1] + 1.0
  ns = lax.rsqrt(deg_s)
  nd = lax.rsqrt(deg_d)
  ns_ref[...] = ns
  nd_ref[...] = nd
  xw = jnp.dot(xp_ref[...], w1_ref[...], preferred_element_type=F32)
  hw1_ref[...] = xw * ns


def _merge_halves(part_ref):
  return jnp.concatenate(
      [part_ref[0], part_ref[1], part_ref[2], part_ref[3]], axis=0)


def _tc2_body(part_ref, hw1_ref, ns_ref, nd_ref, b1_ref, w2_ref, hw2_ref):
  m = _merge_halves(part_ref) + hw1_ref[...]
  h = jax.nn.relu(m * nd_ref[...] + b1_ref[...][None, :])
  hw2_ref[...] = jnp.dot(h, w2_ref[...],
                         preferred_element_type=F32) * ns_ref[...]


def _tc3_body(part_ref, hw2_ref, nd_ref, b2_ref, wm1_ref, bm1_ref,
              gamma_ref, beta_ref, wm2_ref, bm2_ref, out_ref):
  m = _merge_halves(part_ref) + hw2_ref[...]
  h = jax.nn.relu(m * nd_ref[...] + b2_ref[...][None, :])
  t = jax.nn.relu(
      jnp.dot(h, wm1_ref[...], preferred_element_type=F32)
      + bm1_ref[...][None, :])
  inv = 1.0 / jnp.sqrt(jnp.float32(1.0 + 1e-5))
  t = t * (gamma_ref[...] * inv)[None, :] + beta_ref[...][None, :]
  out_ref[...] = (jnp.dot(t, wm2_ref[...], preferred_element_type=F32)
                  + bm2_ref[...][None, :])


def _tc_call(body, out_shape, *args):
  return pl.pallas_call(body, out_shape=out_shape)(*args)


def kernel(x, edge_index, W1, b1, W2, b2, Wm1, bm1, gamma, beta, Wm2, bm2):
  n, _ = x.shape
  e = edge_index.shape[1]
  np_pad = -(-(n + 1) // 256) * 256
  nb = -(-e // (NW * B))
  nb = -(-nb // 8) * 8
  e_pad = NW * nb * B
  nbatches = e_pad // B

  ei = jnp.pad(edge_index, ((0, 0), (0, e_pad - e)), constant_values=n)
  src2 = ei[0].reshape(-1, B)
  dst2 = ei[1].reshape(-1, B)
  xp = jnp.pad(x, ((0, np_pad - n), (0, 0)))
  zeros_hbm = jnp.zeros((np_pad // (2 * NC) // NS, 128), F32)
  zeros16_hbm = jnp.zeros((np_pad // NS, 16), F32)

  deg_s, deg_d = _deg_call(np_pad, nb, src2, dst2, zeros16_hbm)

  hw1, ns, nd = _tc_call(
      _tc1_body,
      [jax.ShapeDtypeStruct((np_pad, 128), F32),
       jax.ShapeDtypeStruct((np_pad, 1), F32),
       jax.ShapeDtypeStruct((np_pad, 1), F32)],
      xp, deg_s, deg_d, W1)

  part1 = _prop_call(np_pad, nbatches, hw1, src2, dst2, zeros_hbm)

  hw2 = _tc_call(
      _tc2_body,
      jax.ShapeDtypeStruct((np_pad, 128), F32),
      part1, hw1, ns, nd, b1, W2)

  part2 = _prop_call(np_pad, nbatches, hw2, src2, dst2, zeros_hbm)

  out_full = _tc_call(
      _tc3_body,
      jax.ShapeDtypeStruct((np_pad, 2), F32),
      part2, hw2, nd, b2, Wm1, bm1, gamma, beta, Wm2, bm2)

  return out_full[:n]


# speedup vs baseline: 2.8638x; 2.8638x over previous
import functools

import jax
import jax.numpy as jnp
from jax import lax
from jax.experimental import pallas as pl
from jax.experimental.pallas import tpu as pltpu
from jax.experimental.pallas import tpu_sc as plsc

NC = 2
NS = 16
NW = NC * NS
B = 128
RING = 4
PF = 2

F32 = jnp.float32


def _mesh():
  return plsc.VectorSubcoreMesh(core_axis_name="c", subcore_axis_name="s")


def _fill(buf, nrows, ncols, value):
  v = jnp.full((16,), value, F32)
  per_row = ncols // 16

  def body(i, _):
    buf[i // per_row, pl.ds((i % per_row) * 16, 16)] = v
    return 0

  lax.fori_loop(0, nrows * per_row, body, 0)


def _deg_body(nb, rows_per_tile,
              src2, dst2, zeros_hbm, deg_s_out, deg_d_out,
              idx_s, idx_d, ones_v, acc, sem):
  c = lax.axis_index("c")
  s = lax.axis_index("s")
  w = c * NS + s

  _fill(ones_v, B, 16, 1.0)
  base = s * rows_per_tile

  pltpu.sync_copy(src2.at[pl.ds(w * nb, nb)], idx_s)
  pltpu.sync_copy(dst2.at[pl.ds(w * nb, nb)], idx_d)

  for idx, out in ((idx_s, deg_s_out), (idx_d, deg_d_out)):
    pltpu.sync_copy(zeros_hbm, acc.at[pl.ds(base, rows_per_tile)])
    plsc.subcore_barrier()

    def edge_body(j, _, idx=idx):
      pltpu.async_copy(ones_v, acc.at[idx.at[j]], sem, add=True)
      return 0

    lax.fori_loop(0, nb, edge_body, 0)

    def drain_body(j, _, idx=idx):
      pltpu.make_async_copy(ones_v, acc.at[idx.at[0]], sem).wait()
      return 0

    lax.fori_loop(0, nb, drain_body, 0)
    plsc.subcore_barrier()

    pltpu.sync_copy(acc.at[pl.ds(base, rows_per_tile)],
                    out.at[c, pl.ds(base, rows_per_tile)])
    plsc.subcore_barrier()


def _deg_call(np_pad, nb, src2, dst2, zeros_hbm):
  rows_per_tile = np_pad // NS
  out = jax.ShapeDtypeStruct((NC, np_pad, 16), F32)
  f = pl.kernel(
      functools.partial(_deg_body, nb, rows_per_tile),
      out_type=[out, out],
      mesh=_mesh(),
      scratch_types=[
          pltpu.VMEM((nb, B), jnp.int32),
          pltpu.VMEM((nb, B), jnp.int32),
          pltpu.VMEM((B, 16), F32),
          pltpu.VMEM_SHARED((np_pad, 16), F32),
          pltpu.SemaphoreType.DMA,
      ],
  )
  return f(src2, dst2, zeros_hbm)


CH = 8


def _prop_body(nb, rows_per_tile,
               table, src2, dst2, zeros_hbm, part_out,
               idx_s, idx_d, rows, acc, gsem, ssem, isem, dsem):
  c = lax.axis_index("c")
  s = lax.axis_index("s")
  w = c * NS + s
  base = s * rows_per_tile

  pltpu.sync_copy(zeros_hbm, acc.at[pl.ds(base, rows_per_tile)])
  plsc.subcore_barrier()

  def load_chunk(g, p, sync=False):
    src_rows = src2.at[pl.ds(w * nb + g * CH, CH)]
    dst_rows = dst2.at[pl.ds(w * nb + g * CH, CH)]
    sl = pl.ds(p * CH, CH)
    if sync:
      pltpu.sync_copy(src_rows, idx_s.at[sl])
      pltpu.sync_copy(dst_rows, idx_d.at[sl])
    else:
      pltpu.async_copy(src_rows, idx_s.at[sl], isem.at[p])
      pltpu.async_copy(dst_rows, idx_d.at[sl], dsem.at[p])

  def wait_chunk(g, p):
    sl = pl.ds(p * CH, CH)
    pltpu.make_async_copy(src2.at[pl.ds(w * nb + g * CH, CH)],
                          idx_s.at[sl], isem.at[p]).wait()
    pltpu.make_async_copy(dst2.at[pl.ds(w * nb + g * CH, CH)],
                          idx_d.at[sl], dsem.at[p]).wait()

  def gather(cb, r, b):
    pltpu.async_copy(table.at[idx_s.at[cb * CH + r]], rows.at[b],
                     gsem.at[b])

  def gather_wait(cb, r, b):
    pltpu.make_async_copy(table.at[idx_s.at[cb * CH + r]], rows.at[b],
                          gsem.at[b]).wait()

  def scatter(cb, r, b):
    pltpu.async_copy(rows.at[b], acc.at[idx_d.at[cb * CH + r]],
                     ssem.at[b], add=True)

  def scatter_wait(b):
    pltpu.make_async_copy(rows.at[b], acc.at[idx_d.at[0]],
                          ssem.at[b]).wait()

  load_chunk(0, 0, sync=True)
  gather(0, 0, 0)
  gather(0, 1, 1)

  nchunks = nb // CH

  def outer(gg, _):
    for p in range(2):
      g = 2 * gg + p
      load_chunk(g + 1, p ^ 1)
      for pos in range(CH):
        b = pos % 2
        if pos == CH - 2:
          wait_chunk(g + 1, p ^ 1)
        gather_wait(p, pos, b)
        scatter(p, pos, b)
        scatter_wait(b)
        np2 = pos + 2
        gather((p ^ 1) if np2 >= CH else p, np2 % CH, b)
    return 0

  lax.fori_loop(0, nchunks // 2, outer, 0)
  gather_wait(0, 0, 0)
  gather_wait(0, 1, 1)
  plsc.subcore_barrier()

  pltpu.sync_copy(acc.at[pl.ds(base, rows_per_tile)],
                  part_out.at[c, pl.ds(base, rows_per_tile)])


def _prop_call(np_pad, nb, table, src2, dst2, zeros_hbm):
  rows_per_tile = np_pad // NS
  f = pl.kernel(
      functools.partial(_prop_body, nb, rows_per_tile),
      out_type=jax.ShapeDtypeStruct((NC, np_pad, 128), F32),
      mesh=_mesh(),
      scratch_types=[
          pltpu.VMEM((2 * CH, B), jnp.int32),
          pltpu.VMEM((2 * CH, B), jnp.int32),
          pltpu.VMEM((2, B, 128), F32),
          pltpu.VMEM_SHARED((np_pad, 128), F32),
          pltpu.SemaphoreType.DMA((2,)),
          pltpu.SemaphoreType.DMA((2,)),
          pltpu.SemaphoreType.DMA((2,)),
          pltpu.SemaphoreType.DMA((2,)),
      ],
  )
  return f(table, src2, dst2, zeros_hbm)


def _tc1_body(xp_ref, ds_ref, dd_ref, w1_ref, hw1_ref, ns_ref, nd_ref):
  deg_s = ds_ref[0, :, 0:1] + ds_ref[1, :, 0:1] + 1.0
  deg_d = dd_ref[0, :, 0:1] + dd_ref[1, :, 0:1] + 1.0
  ns = lax.rsqrt(deg_s)
  nd = lax.rsqrt(deg_d)
  ns_ref[...] = ns
  nd_ref[...] = nd
  xw = jnp.dot(xp_ref[...], w1_ref[...], preferred_element_type=F32)
  hw1_ref[...] = xw * ns


def _merge_halves(part_ref):
  return part_ref[0] + part_ref[1]


def _tc2_body(part_ref, hw1_ref, ns_ref, nd_ref, b1_ref, w2_ref, hw2_ref):
  m = _merge_halves(part_ref) + hw1_ref[...]
  h = jax.nn.relu(m * nd_ref[...] + b1_ref[...][None, :])
  hw2_ref[...] = jnp.dot(h, w2_ref[...],
                         preferred_element_type=F32) * ns_ref[...]


def _tc3_body(part_ref, hw2_ref, nd_ref, b2_ref, wm1_ref, bm1_ref,
              gamma_ref, beta_ref, wm2_ref, bm2_ref, out_ref):
  m = _merge_halves(part_ref) + hw2_ref[...]
  h = jax.nn.relu(m * nd_ref[...] + b2_ref[...][None, :])
  t = jax.nn.relu(
      jnp.dot(h, wm1_ref[...], preferred_element_type=F32)
      + bm1_ref[...][None, :])
  inv = 1.0 / jnp.sqrt(jnp.float32(1.0 + 1e-5))
  t = t * (gamma_ref[...] * inv)[None, :] + beta_ref[...][None, :]
  out_ref[...] = (jnp.dot(t, wm2_ref[...], preferred_element_type=F32)
                  + bm2_ref[...][None, :])


def _tc_call(body, out_shape, *args):
  return pl.pallas_call(body, out_shape=out_shape)(*args)


def kernel(x, edge_index, W1, b1, W2, b2, Wm1, bm1, gamma, beta, Wm2, bm2):
  n, _ = x.shape
  e = edge_index.shape[1]
  np_pad = -(-(n + 1) // 256) * 256
  nb = -(-e // (NW * B))
  nb = -(-nb // 8) * 8
  e_pad = NW * nb * B
  nbatches = e_pad // B

  ei = jnp.pad(edge_index, ((0, 0), (0, e_pad + 16 * B - e)),
               constant_values=n)
  src2 = ei[0].reshape(-1, B)
  dst2 = ei[1].reshape(-1, B)
  xp = jnp.pad(x, ((0, np_pad - n), (0, 0)))
  zeros_hbm = jnp.zeros((np_pad // NS, 128), F32)
  zeros16_hbm = jnp.zeros((np_pad // NS, 16), F32)

  deg_s, deg_d = _deg_call(np_pad, nb, src2, dst2, zeros16_hbm)

  hw1, ns, nd = _tc_call(
      _tc1_body,
      [jax.ShapeDtypeStruct((np_pad, 128), F32),
       jax.ShapeDtypeStruct((np_pad, 1), F32),
       jax.ShapeDtypeStruct((np_pad, 1), F32)],
      xp, deg_s, deg_d, W1)

  part1 = _prop_call(np_pad, nb, hw1, src2, dst2, zeros_hbm)

  hw2 = _tc_call(
      _tc2_body,
      jax.ShapeDtypeStruct((np_pad, 128), F32),
      part1, hw1, ns, nd, b1, W2)

  part2 = _prop_call(np_pad, nb, hw2, src2, dst2, zeros_hbm)

  out_full = _tc_call(
      _tc3_body,
      jax.ShapeDtypeStruct((np_pad, 2), F32),
      part2, hw2, nd, b2, Wm1, bm1, gamma, beta, Wm2, bm2)

  return out_full[:n]


# speedup vs baseline: 2.8883x; 1.0086x over previous
import functools

import jax
import jax.numpy as jnp
from jax import lax
from jax.experimental import pallas as pl
from jax.experimental.pallas import tpu as pltpu
from jax.experimental.pallas import tpu_sc as plsc

NC = 2
NS = 16
NW = NC * NS
B = 128
RING = 4
PF = 2

F32 = jnp.float32


def _mesh():
  return plsc.VectorSubcoreMesh(core_axis_name="c", subcore_axis_name="s")


def _fill(buf, nrows, ncols, value):
  v = jnp.full((16,), value, F32)
  per_row = ncols // 16

  def body(i, _):
    buf[i // per_row, pl.ds((i % per_row) * 16, 16)] = v
    return 0

  lax.fori_loop(0, nrows * per_row, body, 0)


def _deg_body(nb, rows_per_tile, nb_real,
              src2, dst2, zeros_hbm, deg_s_out, deg_d_out,
              idx_s, idx_d, ones_v, acc, sem):
  c = lax.axis_index("c")
  s = lax.axis_index("s")
  w = c * NS + s
  limit = jnp.maximum(0, jnp.minimum(nb, nb_real - w * nb))

  _fill(ones_v, B, 16, 1.0)
  base = s * rows_per_tile

  pltpu.sync_copy(src2.at[pl.ds(w * nb, nb)], idx_s)
  pltpu.sync_copy(dst2.at[pl.ds(w * nb, nb)], idx_d)

  for idx, out in ((idx_s, deg_s_out), (idx_d, deg_d_out)):
    pltpu.sync_copy(zeros_hbm, acc.at[pl.ds(base, rows_per_tile)])
    plsc.subcore_barrier()

    def edge_body(j, _, idx=idx):
      @pl.when(j < limit)
      def _():
        pltpu.async_copy(ones_v, acc.at[idx.at[j]], sem, add=True)
      return 0

    lax.fori_loop(0, nb, edge_body, 0)

    def drain_body(j, _, idx=idx):
      @pl.when(j < limit)
      def _():
        pltpu.make_async_copy(ones_v, acc.at[idx.at[0]], sem).wait()
      return 0

    lax.fori_loop(0, nb, drain_body, 0)
    plsc.subcore_barrier()

    pltpu.sync_copy(acc.at[pl.ds(base, rows_per_tile)],
                    out.at[c, pl.ds(base, rows_per_tile)])
    plsc.subcore_barrier()


def _deg_call(np_pad, nb, nb_real, src2, dst2, zeros_hbm):
  rows_per_tile = np_pad // NS
  out = jax.ShapeDtypeStruct((NC, np_pad, 16), F32)
  f = pl.kernel(
      functools.partial(_deg_body, nb, rows_per_tile, nb_real),
      out_type=[out, out],
      mesh=_mesh(),
      scratch_types=[
          pltpu.VMEM((nb, B), jnp.int32),
          pltpu.VMEM((nb, B), jnp.int32),
          pltpu.VMEM((B, 16), F32),
          pltpu.VMEM_SHARED((np_pad, 16), F32),
          pltpu.SemaphoreType.DMA,
      ],
  )
  return f(src2, dst2, zeros_hbm)


CH = 8


def _prop_body(nb, rows_per_tile, nb_real,
               table, src2, dst2, zeros_hbm, part_out,
               idx_s, idx_d, rows, acc, gsem, ssem, isem, dsem):
  c = lax.axis_index("c")
  s = lax.axis_index("s")
  w = c * NS + s
  base = s * rows_per_tile
  limit = jnp.maximum(0, jnp.minimum(nb, nb_real - w * nb))

  pltpu.sync_copy(zeros_hbm, acc.at[pl.ds(base, rows_per_tile)])
  plsc.subcore_barrier()

  def load_chunk(g, p, sync=False):
    src_rows = src2.at[pl.ds(w * nb + g * CH, CH)]
    dst_rows = dst2.at[pl.ds(w * nb + g * CH, CH)]
    sl = pl.ds(p * CH, CH)
    if sync:
      pltpu.sync_copy(src_rows, idx_s.at[sl])
      pltpu.sync_copy(dst_rows, idx_d.at[sl])
    else:
      pltpu.async_copy(src_rows, idx_s.at[sl], isem.at[p])
      pltpu.async_copy(dst_rows, idx_d.at[sl], dsem.at[p])

  def wait_chunk(g, p):
    sl = pl.ds(p * CH, CH)
    pltpu.make_async_copy(src2.at[pl.ds(w * nb + g * CH, CH)],
                          idx_s.at[sl], isem.at[p]).wait()
    pltpu.make_async_copy(dst2.at[pl.ds(w * nb + g * CH, CH)],
                          idx_d.at[sl], dsem.at[p]).wait()

  def gather(cb, r, b):
    pltpu.async_copy(table.at[idx_s.at[cb * CH + r]], rows.at[b],
                     gsem.at[b])

  def gather_wait(cb, r, b):
    pltpu.make_async_copy(table.at[idx_s.at[cb * CH + r]], rows.at[b],
                          gsem.at[b]).wait()

  def scatter(cb, r, b):
    pltpu.async_copy(rows.at[b], acc.at[idx_d.at[cb * CH + r]],
                     ssem.at[b], add=True)

  def scatter_wait(b):
    pltpu.make_async_copy(rows.at[b], acc.at[idx_d.at[0]],
                          ssem.at[b]).wait()

  load_chunk(0, 0, sync=True)
  gather(0, 0, 0)
  gather(0, 1, 1)

  nchunks = nb // CH

  def outer(gg, _):
    for p in range(2):
      g = 2 * gg + p
      load_chunk(g + 1, p ^ 1)
      for pos in range(CH):
        b = pos % 2
        if pos == CH - 2:
          wait_chunk(g + 1, p ^ 1)
        gather_wait(p, pos, b)

        @pl.when(g * CH + pos < limit)
        def _():
          scatter(p, pos, b)
          scatter_wait(b)
        np2 = pos + 2
        gather((p ^ 1) if np2 >= CH else p, np2 % CH, b)
    return 0

  lax.fori_loop(0, nchunks // 2, outer, 0)
  gather_wait(0, 0, 0)
  gather_wait(0, 1, 1)
  plsc.subcore_barrier()

  pltpu.sync_copy(acc.at[pl.ds(base, rows_per_tile)],
                  part_out.at[c, pl.ds(base, rows_per_tile)])


def _prop_call(np_pad, nb, nb_real, table, src2, dst2, zeros_hbm):
  rows_per_tile = np_pad // NS
  f = pl.kernel(
      functools.partial(_prop_body, nb, rows_per_tile, nb_real),
      out_type=jax.ShapeDtypeStruct((NC, np_pad, 128), F32),
      mesh=_mesh(),
      scratch_types=[
          pltpu.VMEM((2 * CH, B), jnp.int32),
          pltpu.VMEM((2 * CH, B), jnp.int32),
          pltpu.VMEM((2, B, 128), F32),
          pltpu.VMEM_SHARED((np_pad, 128), F32),
          pltpu.SemaphoreType.DMA((2,)),
          pltpu.SemaphoreType.DMA((2,)),
          pltpu.SemaphoreType.DMA((2,)),
          pltpu.SemaphoreType.DMA((2,)),
      ],
  )
  return f(table, src2, dst2, zeros_hbm)


def _tc1_body(xp_ref, ds_ref, dd_ref, w1_ref, hw1_ref, ns_ref, nd_ref):
  deg_s = ds_ref[0, :, 0:1] + ds_ref[1, :, 0:1] + 1.0
  deg_d = dd_ref[0, :, 0:1] + dd_ref[1, :, 0:1] + 1.0
  ns = lax.rsqrt(deg_s)
  nd = lax.rsqrt(deg_d)
  ns_ref[...] = ns
  nd_ref[...] = nd
  xw = jnp.dot(xp_ref[...], w1_ref[...], preferred_element_type=F32)
  hw1_ref[...] = xw * ns


def _merge_halves(part_ref):
  return part_ref[0] + part_ref[1]


def _tc2_body(part_ref, hw1_ref, ns_ref, nd_ref, b1_ref, w2_ref, hw2_ref):
  m = _merge_halves(part_ref) + hw1_ref[...]
  h = jax.nn.relu(m * nd_ref[...] + b1_ref[...][None, :])
  hw2_ref[...] = jnp.dot(h, w2_ref[...],
                         preferred_element_type=F32) * ns_ref[...]


def _tc3_body(part_ref, hw2_ref, nd_ref, b2_ref, wm1_ref, bm1_ref,
              gamma_ref, beta_ref, wm2_ref, bm2_ref, out_ref):
  m = _merge_halves(part_ref) + hw2_ref[...]
  h = jax.nn.relu(m * nd_ref[...] + b2_ref[...][None, :])
  t = jax.nn.relu(
      jnp.dot(h, wm1_ref[...], preferred_element_type=F32)
      + bm1_ref[...][None, :])
  inv = 1.0 / jnp.sqrt(jnp.float32(1.0 + 1e-5))
  t = t * (gamma_ref[...] * inv)[None, :] + beta_ref[...][None, :]
  out_ref[...] = (jnp.dot(t, wm2_ref[...], preferred_element_type=F32)
                  + bm2_ref[...][None, :])


def _tc_call(body, out_shape, *args):
  return pl.pallas_call(body, out_shape=out_shape)(*args)


def kernel(x, edge_index, W1, b1, W2, b2, Wm1, bm1, gamma, beta, Wm2, bm2):
  n, _ = x.shape
  e = edge_index.shape[1]
  np_pad = -(-(n + 1) // 256) * 256
  nb = -(-e // (NW * B))
  nb = -(-nb // 8) * 8
  e_pad = NW * nb * B
  nbatches = e_pad // B

  ei = jnp.pad(edge_index, ((0, 0), (0, e_pad + 16 * B - e)),
               constant_values=n)
  src2 = ei[0].reshape(-1, B)
  dst2 = ei[1].reshape(-1, B)
  xp = jnp.pad(x, ((0, np_pad - n), (0, 0)))
  zeros_hbm = jnp.zeros((np_pad // NS, 128), F32)
  zeros16_hbm = jnp.zeros((np_pad // NS, 16), F32)

  nb_real = -(-e // B)
  deg_s, deg_d = _deg_call(np_pad, nb, nb_real, src2, dst2, zeros16_hbm)

  hw1, ns, nd = _tc_call(
      _tc1_body,
      [jax.ShapeDtypeStruct((np_pad, 128), F32),
       jax.ShapeDtypeStruct((np_pad, 1), F32),
       jax.ShapeDtypeStruct((np_pad, 1), F32)],
      xp, deg_s, deg_d, W1)

  part1 = _prop_call(np_pad, nb, nb_real, hw1, src2, dst2, zeros_hbm)

  hw2 = _tc_call(
      _tc2_body,
      jax.ShapeDtypeStruct((np_pad, 128), F32),
      part1, hw1, ns, nd, b1, W2)

  part2 = _prop_call(np_pad, nb, nb_real, hw2, src2, dst2, zeros_hbm)

  out_full = _tc_call(
      _tc3_body,
      jax.ShapeDtypeStruct((np_pad, 2), F32),
      part2, hw2, nd, b2, Wm1, bm1, gamma, beta, Wm2, bm2)

  return out_full[:n]
